# baseline (speedup 1.0000x reference)
import functools

import jax
import jax.numpy as jnp
from jax.experimental import pallas as pl
from jax.experimental.pallas import tpu as pltpu


def _round_up(a, m):
    return ((a + m - 1) // m) * m


def _pad2(a, rows, cols, dtype):
    a = a.astype(dtype)
    if a.shape == (rows, cols):
        return a
    return jnp.zeros((rows, cols), dtype).at[: a.shape[0], : a.shape[1]].set(a)


def _gcn_body(x_ref, w_ref, adj_ref, b_ref, o_ref, sup_ref, *, tk):
    i = pl.program_id(0)
    k = pl.program_id(1)
    start = pl.multiple_of(k * tk, tk)

    @pl.when(i == 0)
    def _compute_support_block():
        sup_ref[pl.ds(start, tk), :] = jnp.dot(
            x_ref[pl.ds(start, tk), :], w_ref[...],
            preferred_element_type=jnp.float32,
        )

    @pl.when(k == 0)
    def _init_out():
        o_ref[...] = jnp.broadcast_to(b_ref[...], o_ref.shape)

    o_ref[...] += jnp.dot(
        adj_ref[...], sup_ref[pl.ds(start, tk), :],
        preferred_element_type=jnp.float32,
    )


def kernel(x, adj, weight, bias):
    x = jnp.squeeze(x)
    adj = jnp.squeeze(adj)
    N, Fin = x.shape
    Fout = weight.shape[1]
    if bias is None:
        bias = jnp.zeros((Fout,), jnp.float32)

    n_pad = _round_up(N, 512)
    fin_pad = _round_up(Fin, 128)
    fout_pad = _round_up(Fout, 128)

    x_p = _pad2(x, n_pad, fin_pad, jnp.float32)
    w_p = _pad2(weight, fin_pad, fout_pad, jnp.float32)
    adj_p = _pad2(adj, n_pad, n_pad, jnp.float32)
    b_p = _pad2(bias.reshape(1, Fout), 1, fout_pad, jnp.float32)

    tm = 512
    tk = 1024
    body = functools.partial(_gcn_body, tk=tk)
    out_p = pl.pallas_call(
        body,
        out_shape=jax.ShapeDtypeStruct((n_pad, fout_pad), jnp.float32),
        grid=(n_pad // tm, n_pad // tk),
        in_specs=[
            pl.BlockSpec((n_pad, fin_pad), lambda i, k: (0, 0)),
            pl.BlockSpec((fin_pad, fout_pad), lambda i, k: (0, 0)),
            pl.BlockSpec((tm, tk), lambda i, k: (i, k)),
            pl.BlockSpec((1, fout_pad), lambda i, k: (0, 0)),
        ],
        out_specs=pl.BlockSpec((tm, fout_pad), lambda i, k: (i, 0)),
        scratch_shapes=[pltpu.VMEM((n_pad, fout_pad), jnp.float32)],
        compiler_params=pltpu.CompilerParams(
            dimension_semantics=("arbitrary", "arbitrary"),
            vmem_limit_bytes=64 * 1024 * 1024,
        ),
        cost_estimate=pl.CostEstimate(
            flops=2 * n_pad * fout_pad * (n_pad + fin_pad),
            transcendentals=0,
            bytes_accessed=4 * n_pad * n_pad
            + 4 * n_pad * fin_pad
            + 4 * fin_pad * fout_pad
            + 4 * fout_pad
            + 4 * n_pad * fout_pad,
        ),
    )(x_p, w_p, adj_p, b_p)

    return out_p[:N, :Fout]


# speedup vs baseline: 1.5277x; 1.5277x over previous
import functools

import jax
import jax.numpy as jnp
from jax.experimental import pallas as pl
from jax.experimental.pallas import tpu as pltpu


def _round_up(a, m):
    return ((a + m - 1) // m) * m


def _pad2(a, rows, cols, dtype):
    a = a.astype(dtype)
    if a.shape == (rows, cols):
        return a
    return jnp.zeros((rows, cols), dtype).at[: a.shape[0], : a.shape[1]].set(a)


def _gcn_body(x_ref, w_ref, adj_hbm, b_ref, o_ref, sup_ref, abuf, sem,
              *, tm, nblk):
    i = pl.program_id(0)

    def copy(blk, slot):
        pltpu.make_async_copy(
            adj_hbm.at[pl.ds(blk * tm, tm), :], abuf.at[slot], sem.at[slot]
        ).start()

    @pl.when(i == 0)
    def _prologue():
        copy(0, 0)
        if nblk > 1:
            copy(1, 1)
        sup_ref[...] = jnp.dot(
            x_ref[...], w_ref[...], preferred_element_type=jnp.float32
        )

    @pl.when((i > 0) & (i + 1 < nblk))
    def _prefetch_next():
        copy(i + 1, (i + 1) % 2)

    slot = i % 2
    pltpu.make_async_copy(
        adj_hbm.at[pl.ds(0, tm), :], abuf.at[slot], sem.at[slot]
    ).wait()
    o_ref[...] = (
        jnp.dot(abuf[slot], sup_ref[...], preferred_element_type=jnp.float32)
        + b_ref[...]
    )


def kernel(x, adj, weight, bias):
    x = jnp.squeeze(x)
    adj = jnp.squeeze(adj)
    N, Fin = x.shape
    Fout = weight.shape[1]
    if bias is None:
        bias = jnp.zeros((Fout,), jnp.float32)

    n_pad = _round_up(N, 512)
    fin_pad = _round_up(Fin, 128)
    fout_pad = _round_up(Fout, 128)

    x_p = _pad2(x, n_pad, fin_pad, jnp.float32)
    w_p = _pad2(weight, fin_pad, fout_pad, jnp.float32)
    adj_p = _pad2(adj, n_pad, n_pad, jnp.float32)
    b_p = _pad2(bias.reshape(1, Fout), 1, fout_pad, jnp.float32)

    tm = 512
    nblk = n_pad // tm
    body = functools.partial(_gcn_body, tm=tm, nblk=nblk)
    out_p = pl.pallas_call(
        body,
        out_shape=jax.ShapeDtypeStruct((n_pad, fout_pad), jnp.float32),
        grid=(nblk,),
        in_specs=[
            pl.BlockSpec((n_pad, fin_pad), lambda i: (0, 0)),
            pl.BlockSpec((fin_pad, fout_pad), lambda i: (0, 0)),
            pl.BlockSpec(memory_space=pltpu.MemorySpace.HBM),
            pl.BlockSpec((1, fout_pad), lambda i: (0, 0)),
        ],
        out_specs=pl.BlockSpec((tm, fout_pad), lambda i: (i, 0)),
        scratch_shapes=[
            pltpu.VMEM((n_pad, fout_pad), jnp.float32),
            pltpu.VMEM((2, tm, n_pad), jnp.float32),
            pltpu.SemaphoreType.DMA((2,)),
        ],
        compiler_params=pltpu.CompilerParams(
            dimension_semantics=("arbitrary",),
            vmem_limit_bytes=64 * 1024 * 1024,
        ),
        cost_estimate=pl.CostEstimate(
            flops=2 * n_pad * fout_pad * (n_pad + fin_pad),
            transcendentals=0,
            bytes_accessed=4 * n_pad * n_pad
            + 4 * n_pad * fin_pad
            + 4 * fin_pad * fout_pad
            + 4 * fout_pad
            + 4 * n_pad * fout_pad,
        ),
    )(x_p, w_p, adj_p, b_p)

    return out_p[:N, :Fout]
